# baseline (speedup 1.0000x reference)
import dataclasses
import functools

import jax
import jax.numpy as jnp
from jax import lax
from jax.experimental import pallas as pl
from jax.experimental.pallas import tpu as pltpu
from jax.experimental.pallas import tpu_sc as plsc

N = 10000
E = 320000
R = 4
D = 128
NC, NS = 2, 16
NW = NC * NS

CH_P = 80
EPW = E // NW
NCH_W = EPW // CH_P
EPS = E // NS
NCH_CNT = EPS // CH_P
CNT_PAD = 10240
CNT_SZ = R * CNT_PAD

N_PAD = 10240

CH_E = 96
NCH_E = 108
EPW_PAD = NCH_E * CH_E
E_PAD = NW * EPW_PAD
CNT_CH = 128

_mesh = plsc.VectorSubcoreMesh(core_axis_name="c", subcore_axis_name="s")

_sc_params = pltpu.CompilerParams()
if "needs_layout_passes" in pltpu.CompilerParams.__dataclass_fields__:
    _sc_params = dataclasses.replace(_sc_params, needs_layout_passes=False)


DUMP = 10200
HALF = 10000
NRW = HALF // CNT_CH + 1


@functools.partial(
    pl.kernel,
    out_type=(
        jax.ShapeDtypeStruct((E_PAD,), jnp.float32),
        jax.ShapeDtypeStruct((E_PAD,), jnp.int32),
        jax.ShapeDtypeStruct((E_PAD,), jnp.int32),
    ),
    mesh=_mesh,
    scratch_types=[
        pltpu.VMEM_SHARED((CNT_SZ,), jnp.float32),
        pltpu.VMEM((CNT_SZ,), jnp.float32),
        pltpu.VMEM((EPW_PAD,), jnp.int32),
        pltpu.VMEM((EPW_PAD,), jnp.int32),
        pltpu.VMEM((EPW_PAD,), jnp.int32),
        pltpu.VMEM((NRW + 1, CNT_CH), jnp.int32),
        pltpu.VMEM((CNT_CH,), jnp.float32),
        pltpu.VMEM((EPW_PAD,), jnp.float32),
        pltpu.VMEM((EPW_PAD,), jnp.int32),
        pltpu.SemaphoreType.DMA,
    ],
    compiler_params=_sc_params,
)
def _prep_kernel(src_h, dst_h, typ_h, w_h, g_h, do_h,
                 cnt_sh, cnt_loc, e1, e2, es, idx2, ones, w_all, g_all, csem):
    c = lax.axis_index("c")
    s = lax.axis_index("s")
    z16f = jnp.zeros((16,), jnp.float32)
    z16i = jnp.zeros((16,), jnp.int32)
    one16 = jnp.ones((16,), jnp.float32)
    d16 = jnp.full((16,), DUMP, jnp.int32)

    @pl.loop(0, CNT_SZ, step=16)
    def _(i):
        cnt_loc[pl.ds(i, 16)] = z16f

    seg = CNT_SZ // NS
    pltpu.sync_copy(cnt_loc.at[pl.ds(s * seg, seg)], cnt_sh.at[pl.ds(s * seg, seg)])
    for j in range(CNT_CH // 16):
        ones[pl.ds(j * 16, 16)] = one16
    for j in range((EPW_PAD - HALF) // 16):
        e1[pl.ds(HALF + j * 16, 16)] = z16i
        e2[pl.ds(HALF + j * 16, 16)] = d16
    plsc.subcore_barrier()

    @pl.loop(0, 2)
    def _(p):
        base = s * EPS + p * HALF
        pltpu.sync_copy(typ_h.at[pl.ds(base, HALF)], e1.at[pl.ds(0, HALF)])
        pltpu.sync_copy(dst_h.at[pl.ds(base, HALF)], e2.at[pl.ds(0, HALF)])

        @pl.loop(0, NRW)
        def _(r):
            for j in range(CNT_CH // 16):
                sl = pl.ds(r * CNT_CH + j * 16, 16)
                idx2[r, pl.ds(j * 16, 16)] = e1[sl] * CNT_PAD + e2[sl]

        @pl.loop(0, NRW - 7, step=8)
        def _(ci):
            hs = [pltpu.async_copy(ones, cnt_sh.at[idx2.at[ci + k]], csem,
                                   add=True) for k in range(8)]
            for h in hs:
                h.wait()

        hs = [pltpu.async_copy(ones, cnt_sh.at[idx2.at[NRW - 7 + k]], csem,
                               add=True) for k in range(7)]
        for h in hs:
            h.wait()

    plsc.subcore_barrier()
    pltpu.sync_copy(cnt_sh, cnt_loc)

    wid = s * NC + c
    bi = wid * EPW
    pltpu.sync_copy(typ_h.at[pl.ds(bi, HALF)], e1.at[pl.ds(0, HALF)])
    pltpu.sync_copy(dst_h.at[pl.ds(bi, HALF)], e2.at[pl.ds(0, HALF)])
    pltpu.sync_copy(src_h.at[pl.ds(bi, HALF)], es.at[pl.ds(0, HALF)])

    @pl.loop(0, EPW_PAD, step=16)
    def _(i):
        sl = pl.ds(i, 16)
        t16 = e1[sl]
        cidx = t16 * CNT_PAD + e2[sl]
        cv = plsc.load_gather(cnt_loc, [cidx])
        w_all[sl] = 1.0 / jnp.maximum(cv, 1.0)
        g_all[sl] = es[sl] * R + t16

    for j in range((EPW_PAD - HALF) // 16):
        sl = pl.ds(HALF + j * 16, 16)
        w_all[sl] = z16f
        g_all[sl] = z16i
        e2[sl] = z16i

    bo = wid * EPW_PAD
    pltpu.sync_copy(w_all, w_h.at[pl.ds(bo, EPW_PAD)])
    pltpu.sync_copy(g_all, g_h.at[pl.ds(bo, EPW_PAD)])
    pltpu.sync_copy(e2, do_h.at[pl.ds(bo, EPW_PAD)])


@functools.partial(
    pl.kernel,
    out_type=jax.ShapeDtypeStruct((NC, N_PAD, D), jnp.float32),
    mesh=_mesh,
    scratch_types=[
        pltpu.VMEM_SHARED((N_PAD, D), jnp.float32),
        [pltpu.VMEM((CH_E,), jnp.int32) for _ in range(4)],
        [pltpu.VMEM((CH_E,), jnp.float32) for _ in range(4)],
        [pltpu.VMEM((CH_E,), jnp.int32) for _ in range(4)],
        [pltpu.VMEM((CH_E, D), jnp.float32) for _ in range(3)],
        [pltpu.SemaphoreType.DMA for _ in range(4)],
        [pltpu.SemaphoreType.DMA for _ in range(3)],
    ],
    compiler_params=_sc_params,
)
def _edge_kernel(y_h, g_h, w_h, d_h, z_h, o_h, acc_sh,
                 gix, wv, dix, rows, isem, gsem):
    c = lax.axis_index("c")
    s = lax.axis_index("s")
    wid = s * NC + c
    npt = N_PAD // NS

    pltpu.sync_copy(z_h.at[pl.ds(s * npt, npt)], acc_sh.at[pl.ds(s * npt, npt)])

    def issue_idx(ch, q):
        pltpu.async_copy(g_h.at[wid, ch], gix[q], isem[q])
        pltpu.async_copy(w_h.at[wid, ch], wv[q], isem[q])
        pltpu.async_copy(d_h.at[wid, ch], dix[q], isem[q])

    def wait_idx(ch, q):
        pltpu.make_async_copy(g_h.at[wid, ch], gix[q], isem[q]).wait()
        pltpu.make_async_copy(w_h.at[wid, ch], wv[q], isem[q]).wait()
        pltpu.make_async_copy(d_h.at[wid, ch], dix[q], isem[q]).wait()

    def scale(rb, wvb):
        @pl.loop(0, CH_E)
        def _(e):
            wb = plsc.load_gather(wvb, [jnp.zeros((16,), jnp.int32) + e])
            for j in range(D // 16):
                sl = (e, pl.ds(j * 16, 16))
                rb[sl] = rb[sl] * wb

    pltpu.sync_copy(g_h.at[wid, 0], gix[0])
    pltpu.sync_copy(w_h.at[wid, 0], wv[0])
    pltpu.sync_copy(d_h.at[wid, 0], dix[0])
    pltpu.sync_copy(g_h.at[wid, 1], gix[1])
    pltpu.sync_copy(w_h.at[wid, 1], wv[1])
    pltpu.sync_copy(d_h.at[wid, 1], dix[1])
    pltpu.async_copy(y_h.at[gix[0]], rows[0], gsem[0])
    pltpu.async_copy(y_h.at[gix[1]], rows[1], gsem[1])
    issue_idx(2, 2)
    plsc.subcore_barrier()

    @pl.loop(0, NCH_E, step=12)
    def _(cm):
        for k in range(12):
            ch = cm + k
            r = k % 3
            q = k % 4
            r2 = (k + 2) % 3
            q2 = (k + 2) % 4
            q3 = (k + 3) % 4
            pltpu.make_async_copy(y_h.at[gix[q]], rows[r], gsem[r]).wait()

            @pl.when(ch + 3 < NCH_E)
            def _():
                issue_idx(ch + 3, q3)

            @pl.when(ch + 2 < NCH_E)
            def _():
                wait_idx(ch + 2, q2)
                pltpu.async_copy(y_h.at[gix[q2]], rows[r2], gsem[r2])

            scale(rows[r], wv[q])
            pltpu.sync_copy(rows[r], acc_sh.at[dix[q]], add=True)

    plsc.subcore_barrier()
    pltpu.sync_copy(acc_sh.at[pl.ds(s * npt, npt)], o_h.at[c, pl.ds(s * npt, npt)])


BN = 1000


def _mm1_body(x_ref, w2_ref, wr_ref, b_ref, y_ref, r_ref):
    xb = x_ref[...]
    y_ref[...] = jnp.dot(xb, w2_ref[...], preferred_element_type=jnp.float32)
    r_ref[...] = jnp.dot(xb, wr_ref[...], preferred_element_type=jnp.float32) + b_ref[...]


def _mm2_body(r0_ref, p0_ref, p1_ref, w2_ref, wr_ref, b_ref, y_ref, r_ref):
    hb = jnp.maximum(r0_ref[...] + p0_ref[...] + p1_ref[...], 0.0)
    y_ref[...] = jnp.dot(hb, w2_ref[...], preferred_element_type=jnp.float32)
    r_ref[...] = jnp.dot(hb, wr_ref[...], preferred_element_type=jnp.float32) + b_ref[...]


def _fin_body(r_ref, p0_ref, p1_ref, o_ref):
    i = pl.program_id(0)

    @pl.when(i == 0)
    def _():
        o_ref[...] = jnp.zeros_like(o_ref)

    blk = r_ref[...] + p0_ref[...] + p1_ref[...]
    o_ref[...] += jnp.sum(blk, axis=0, keepdims=True)


def _head_body(g_ref, lcw_ref, lcb_ref, emb_ref, d1w_ref, d1b_ref, msg_ref,
               ow_ref, ob_ref, o_ref):
    g = jnp.maximum(jnp.dot(g_ref[...], lcw_ref[...],
                            preferred_element_type=jnp.float32) + lcb_ref[...], 0.0)
    y = jnp.maximum(jnp.dot(emb_ref[...], d1w_ref[...],
                            preferred_element_type=jnp.float32) + d1b_ref[...], 0.0)
    z = jnp.maximum(msg_ref[...], 0.0)
    o_ref[...] = (jnp.dot(g, ow_ref[0], preferred_element_type=jnp.float32)
                  + jnp.dot(y, ow_ref[1], preferred_element_type=jnp.float32)
                  + jnp.dot(z, ow_ref[2], preferred_element_type=jnp.float32)
                  + ob_ref[...])


_nb = pl.BlockSpec((BN, D), lambda i: (i, 0))


def _full(shape):
    return pl.BlockSpec(shape, lambda i: tuple(0 for _ in shape))


_mm1 = pl.pallas_call(
    _mm1_body,
    grid=(N // BN,),
    in_specs=[_nb, _full((D, R * D)), _full((D, D)), _full((1, D))],
    out_specs=[pl.BlockSpec((BN, R * D), lambda i: (i, 0)), _nb],
    out_shape=[jax.ShapeDtypeStruct((N, R * D), jnp.float32),
               jax.ShapeDtypeStruct((N, D), jnp.float32)],
)

_mm2 = pl.pallas_call(
    _mm2_body,
    grid=(N // BN,),
    in_specs=[_nb, _nb, _nb, _full((D, R * D)), _full((D, D)), _full((1, D))],
    out_specs=[pl.BlockSpec((BN, R * D), lambda i: (i, 0)), _nb],
    out_shape=[jax.ShapeDtypeStruct((N, R * D), jnp.float32),
               jax.ShapeDtypeStruct((N, D), jnp.float32)],
)

_fin = pl.pallas_call(
    _fin_body,
    grid=(N // BN,),
    in_specs=[_nb, _nb, _nb],
    out_specs=pl.BlockSpec((1, D), lambda i: (0, 0)),
    out_shape=jax.ShapeDtypeStruct((1, D), jnp.float32),
)

_head = pl.pallas_call(
    _head_body,
    out_shape=jax.ShapeDtypeStruct((1, 2), jnp.float32),
)


@jax.jit
def _run(x, edge_index, edge_type,
         embed, msg, W_rel0, W_root0, b0, W_rel1, W_root1, b1,
         lin_ctg_W, lin_ctg_b, dan1_W, dan1_b, out_W, out_b):
    src = edge_index[0]
    dst = edge_index[1]
    W2_0 = jnp.transpose(W_rel0, (1, 0, 2)).reshape(D, R * D)
    W2_1 = jnp.transpose(W_rel1, (1, 0, 2)).reshape(D, R * D)
    zeros = jnp.zeros((N_PAD, D), jnp.float32)

    w_pad, g_pad, d_pad = _prep_kernel(src, dst, edge_type)
    g2 = g_pad.reshape(NW, NCH_E, CH_E)
    w2 = w_pad.reshape(NW, NCH_E, CH_E)
    d3 = d_pad.reshape(NW, NCH_E, CH_E)

    y0, r0 = _mm1(x, W2_0, W_root0, b0.reshape(1, D))
    p0 = _edge_kernel(y0.reshape(N * R, D), g2, w2, d3, zeros)
    y1, r1 = _mm2(r0, p0[0, :N], p0[1, :N], W2_1, W_root1, b1.reshape(1, D))
    p1 = _edge_kernel(y1.reshape(N * R, D), g2, w2, d3, zeros)
    gvec = _fin(r1, p1[0, :N], p1[1, :N])
    return _head(gvec, lin_ctg_W, lin_ctg_b.reshape(1, -1), embed,
                 dan1_W, dan1_b.reshape(1, -1), msg,
                 out_W.reshape(3, 768, 2), out_b.reshape(1, 2))


def kernel(x, edge_index, edge_type, edge_attr, embed, msg,
           W_rel0, W_root0, b0, W_rel1, W_root1, b1,
           lin_ctg_W, lin_ctg_b, dan1_W, dan1_b, out_W, out_b):
    return _run(x, edge_index, edge_type, embed, msg,
                W_rel0, W_root0, b0, W_rel1, W_root1, b1,
                lin_ctg_W, lin_ctg_b, dan1_W, dan1_b, out_W, out_b)


# speedup vs baseline: 2.0205x; 2.0205x over previous
import dataclasses
import functools

import jax
import jax.numpy as jnp
from jax import lax
from jax.experimental import pallas as pl
from jax.experimental.pallas import tpu as pltpu
from jax.experimental.pallas import tpu_sc as plsc

N = 10000
E = 320000
R = 4
D = 128
NC, NS = 2, 16
NW = NC * NS

CH_P = 80
EPW = E // NW
NCH_W = EPW // CH_P
EPS = E // NS
NCH_CNT = EPS // CH_P
CNT_PAD = 10240
CNT_SZ = R * CNT_PAD

N_PAD = 10240

CH_E = 120
NCH_E = 84
EPW_PAD = NCH_E * CH_E
E_PAD = NW * EPW_PAD
CNT_CH = 128

_mesh = plsc.VectorSubcoreMesh(core_axis_name="c", subcore_axis_name="s")

_sc_params = pltpu.CompilerParams()
if "needs_layout_passes" in pltpu.CompilerParams.__dataclass_fields__:
    _sc_params = dataclasses.replace(_sc_params, needs_layout_passes=False)


DUMP = 10200
HALF = 10000
NRW = HALF // CNT_CH + 1
EBUF = NRW * CNT_CH


@functools.partial(
    pl.kernel,
    out_type=(
        jax.ShapeDtypeStruct((E_PAD,), jnp.float32),
        jax.ShapeDtypeStruct((E_PAD,), jnp.int32),
        jax.ShapeDtypeStruct((E_PAD,), jnp.int32),
    ),
    mesh=_mesh,
    scratch_types=[
        pltpu.VMEM_SHARED((CNT_SZ,), jnp.float32),
        pltpu.VMEM((CNT_SZ,), jnp.float32),
        pltpu.VMEM((EBUF,), jnp.int32),
        pltpu.VMEM((EBUF,), jnp.int32),
        pltpu.VMEM((EBUF,), jnp.int32),
        pltpu.VMEM((NRW + 1, CNT_CH), jnp.int32),
        pltpu.VMEM((CNT_CH,), jnp.float32),
        pltpu.VMEM((EPW_PAD,), jnp.float32),
        pltpu.VMEM((EPW_PAD,), jnp.int32),
        pltpu.SemaphoreType.DMA,
    ],
    compiler_params=_sc_params,
)
def _prep_kernel(src_h, dst_h, typ_h, w_h, g_h, do_h,
                 cnt_sh, cnt_loc, e1, e2, es, idx2, ones, w_all, g_all, csem):
    c = lax.axis_index("c")
    s = lax.axis_index("s")
    z16f = jnp.zeros((16,), jnp.float32)
    z16i = jnp.zeros((16,), jnp.int32)
    one16 = jnp.ones((16,), jnp.float32)
    d16 = jnp.full((16,), DUMP, jnp.int32)

    @pl.loop(0, CNT_SZ, step=16)
    def _(i):
        cnt_loc[pl.ds(i, 16)] = z16f

    seg = CNT_SZ // NS
    pltpu.sync_copy(cnt_loc.at[pl.ds(s * seg, seg)], cnt_sh.at[pl.ds(s * seg, seg)])
    for j in range(CNT_CH // 16):
        ones[pl.ds(j * 16, 16)] = one16
    for j in range((EBUF - HALF) // 16):
        e1[pl.ds(HALF + j * 16, 16)] = z16i
        e2[pl.ds(HALF + j * 16, 16)] = d16
    plsc.subcore_barrier()

    @pl.loop(0, 2)
    def _(p):
        base = s * EPS + p * HALF
        pltpu.sync_copy(typ_h.at[pl.ds(base, HALF)], e1.at[pl.ds(0, HALF)])
        pltpu.sync_copy(dst_h.at[pl.ds(base, HALF)], e2.at[pl.ds(0, HALF)])

        @pl.loop(0, NRW)
        def _(r):
            for j in range(CNT_CH // 16):
                sl = pl.ds(r * CNT_CH + j * 16, 16)
                idx2[r, pl.ds(j * 16, 16)] = e1[sl] * CNT_PAD + e2[sl]

        @pl.loop(0, NRW - 7, step=8)
        def _(ci):
            hs = [pltpu.async_copy(ones, cnt_sh.at[idx2.at[ci + k]], csem,
                                   add=True) for k in range(8)]
            for h in hs:
                h.wait()

        hs = [pltpu.async_copy(ones, cnt_sh.at[idx2.at[NRW - 7 + k]], csem,
                               add=True) for k in range(7)]
        for h in hs:
            h.wait()

    plsc.subcore_barrier()
    pltpu.sync_copy(cnt_sh, cnt_loc)

    wid = s * NC + c
    bi = wid * EPW
    pltpu.sync_copy(typ_h.at[pl.ds(bi, HALF)], e1.at[pl.ds(0, HALF)])
    pltpu.sync_copy(dst_h.at[pl.ds(bi, HALF)], e2.at[pl.ds(0, HALF)])
    pltpu.sync_copy(src_h.at[pl.ds(bi, HALF)], es.at[pl.ds(0, HALF)])

    @pl.loop(0, EPW_PAD, step=16)
    def _(i):
        sl = pl.ds(i, 16)
        t16 = e1[sl]
        cidx = t16 * CNT_PAD + e2[sl]
        cv = plsc.load_gather(cnt_loc, [cidx])
        w_all[sl] = 1.0 / jnp.maximum(cv, 1.0)
        g_all[sl] = es[sl] * R + t16

    for j in range((EPW_PAD - HALF) // 16):
        sl = pl.ds(HALF + j * 16, 16)
        w_all[sl] = z16f
        g_all[sl] = z16i
        e2[sl] = z16i

    bo = wid * EPW_PAD
    pltpu.sync_copy(w_all, w_h.at[pl.ds(bo, EPW_PAD)])
    pltpu.sync_copy(g_all, g_h.at[pl.ds(bo, EPW_PAD)])
    pltpu.sync_copy(e2.at[pl.ds(0, EPW_PAD)], do_h.at[pl.ds(bo, EPW_PAD)])


@functools.partial(
    pl.kernel,
    out_type=jax.ShapeDtypeStruct((NC, N_PAD, D), jnp.float32),
    mesh=_mesh,
    scratch_types=[
        pltpu.VMEM_SHARED((N_PAD, D), jnp.float32),
        [pltpu.VMEM((CH_E,), jnp.int32) for _ in range(4)],
        [pltpu.VMEM((CH_E,), jnp.float32) for _ in range(4)],
        [pltpu.VMEM((CH_E,), jnp.int32) for _ in range(4)],
        [pltpu.VMEM((CH_E, D), jnp.float32) for _ in range(3)],
        [pltpu.SemaphoreType.DMA for _ in range(4)],
        [pltpu.SemaphoreType.DMA for _ in range(3)],
    ],
    compiler_params=_sc_params,
)
def _edge_kernel(y_h, g_h, w_h, d_h, z_h, o_h, acc_sh,
                 gix, wv, dix, rows, isem, gsem):
    c = lax.axis_index("c")
    s = lax.axis_index("s")
    wid = s * NC + c
    npt = N_PAD // NS

    pltpu.sync_copy(z_h.at[pl.ds(s * npt, npt)], acc_sh.at[pl.ds(s * npt, npt)])

    def issue_idx(ch, q):
        pltpu.async_copy(g_h.at[wid, ch], gix[q], isem[q])
        pltpu.async_copy(w_h.at[wid, ch], wv[q], isem[q])
        pltpu.async_copy(d_h.at[wid, ch], dix[q], isem[q])

    def wait_idx(ch, q):
        pltpu.make_async_copy(g_h.at[wid, ch], gix[q], isem[q]).wait()
        pltpu.make_async_copy(w_h.at[wid, ch], wv[q], isem[q]).wait()
        pltpu.make_async_copy(d_h.at[wid, ch], dix[q], isem[q]).wait()

    def scale(rb, wvb):
        @pl.loop(0, CH_E)
        def _(e):
            wb = plsc.load_gather(wvb, [jnp.zeros((16,), jnp.int32) + e])
            for j in range(D // 16):
                sl = (e, pl.ds(j * 16, 16))
                rb[sl] = rb[sl] * wb

    pltpu.sync_copy(g_h.at[wid, 0], gix[0])
    pltpu.sync_copy(w_h.at[wid, 0], wv[0])
    pltpu.sync_copy(d_h.at[wid, 0], dix[0])
    pltpu.sync_copy(g_h.at[wid, 1], gix[1])
    pltpu.sync_copy(w_h.at[wid, 1], wv[1])
    pltpu.sync_copy(d_h.at[wid, 1], dix[1])
    pltpu.async_copy(y_h.at[gix[0]], rows[0], gsem[0])
    pltpu.async_copy(y_h.at[gix[1]], rows[1], gsem[1])
    issue_idx(2, 2)
    plsc.subcore_barrier()

    @pl.loop(0, NCH_E, step=12)
    def _(cm):
        for k in range(12):
            ch = cm + k
            r = k % 3
            q = k % 4
            r2 = (k + 2) % 3
            q2 = (k + 2) % 4
            q3 = (k + 3) % 4
            pltpu.make_async_copy(y_h.at[gix[q]], rows[r], gsem[r]).wait()

            @pl.when(ch + 3 < NCH_E)
            def _():
                issue_idx(ch + 3, q3)

            @pl.when(ch + 2 < NCH_E)
            def _():
                wait_idx(ch + 2, q2)
                pltpu.async_copy(y_h.at[gix[q2]], rows[r2], gsem[r2])

            scale(rows[r], wv[q])
            pltpu.sync_copy(rows[r], acc_sh.at[dix[q]], add=True)

    plsc.subcore_barrier()
    pltpu.sync_copy(acc_sh.at[pl.ds(s * npt, npt)], o_h.at[c, pl.ds(s * npt, npt)])


BN = 1000


def _mm1_body(x_ref, w2_ref, wr_ref, b_ref, y_ref, r_ref):
    xb = x_ref[...]
    y_ref[...] = jnp.dot(xb, w2_ref[...], preferred_element_type=jnp.float32)
    r_ref[...] = jnp.dot(xb, wr_ref[...], preferred_element_type=jnp.float32) + b_ref[...]


def _mm2_body(r0_ref, p0_ref, p1_ref, w2_ref, wr_ref, b_ref, y_ref, r_ref):
    hb = jnp.maximum(r0_ref[...] + p0_ref[...] + p1_ref[...], 0.0)
    y_ref[...] = jnp.dot(hb, w2_ref[...], preferred_element_type=jnp.float32)
    r_ref[...] = jnp.dot(hb, wr_ref[...], preferred_element_type=jnp.float32) + b_ref[...]


def _fin_body(r_ref, p0_ref, p1_ref, o_ref):
    i = pl.program_id(0)

    @pl.when(i == 0)
    def _():
        o_ref[...] = jnp.zeros_like(o_ref)

    blk = r_ref[...] + p0_ref[...] + p1_ref[...]
    o_ref[...] += jnp.sum(blk, axis=0, keepdims=True)


def _head_body(g_ref, lcw_ref, lcb_ref, emb_ref, d1w_ref, d1b_ref, msg_ref,
               ow_ref, ob_ref, o_ref):
    g = jnp.maximum(jnp.dot(g_ref[...], lcw_ref[...],
                            preferred_element_type=jnp.float32) + lcb_ref[...], 0.0)
    y = jnp.maximum(jnp.dot(emb_ref[...], d1w_ref[...],
                            preferred_element_type=jnp.float32) + d1b_ref[...], 0.0)
    z = jnp.maximum(msg_ref[...], 0.0)
    o_ref[...] = (jnp.dot(g, ow_ref[0], preferred_element_type=jnp.float32)
                  + jnp.dot(y, ow_ref[1], preferred_element_type=jnp.float32)
                  + jnp.dot(z, ow_ref[2], preferred_element_type=jnp.float32)
                  + ob_ref[...])


_nb = pl.BlockSpec((BN, D), lambda i: (i, 0))


def _full(shape):
    return pl.BlockSpec(shape, lambda i: tuple(0 for _ in shape))


_mm1 = pl.pallas_call(
    _mm1_body,
    grid=(N // BN,),
    in_specs=[_nb, _full((D, R * D)), _full((D, D)), _full((1, D))],
    out_specs=[pl.BlockSpec((BN, R * D), lambda i: (i, 0)), _nb],
    out_shape=[jax.ShapeDtypeStruct((N, R * D), jnp.float32),
               jax.ShapeDtypeStruct((N, D), jnp.float32)],
)

_mm2 = pl.pallas_call(
    _mm2_body,
    grid=(N // BN,),
    in_specs=[_nb, _nb, _nb, _full((D, R * D)), _full((D, D)), _full((1, D))],
    out_specs=[pl.BlockSpec((BN, R * D), lambda i: (i, 0)), _nb],
    out_shape=[jax.ShapeDtypeStruct((N, R * D), jnp.float32),
               jax.ShapeDtypeStruct((N, D), jnp.float32)],
)

_fin = pl.pallas_call(
    _fin_body,
    grid=(N // BN,),
    in_specs=[_nb, _nb, _nb],
    out_specs=pl.BlockSpec((1, D), lambda i: (0, 0)),
    out_shape=jax.ShapeDtypeStruct((1, D), jnp.float32),
)

_head = pl.pallas_call(
    _head_body,
    out_shape=jax.ShapeDtypeStruct((1, 2), jnp.float32),
)


@jax.jit
def _run(x, edge_index, edge_type,
         embed, msg, W_rel0, W_root0, b0, W_rel1, W_root1, b1,
         lin_ctg_W, lin_ctg_b, dan1_W, dan1_b, out_W, out_b):
    src = edge_index[0]
    dst = edge_index[1]
    W2_0 = jnp.transpose(W_rel0, (1, 0, 2)).reshape(D, R * D)
    W2_1 = jnp.transpose(W_rel1, (1, 0, 2)).reshape(D, R * D)
    zeros = jnp.zeros((N_PAD, D), jnp.float32)

    w_pad, g_pad, d_pad = _prep_kernel(src, dst, edge_type)
    g2 = g_pad.reshape(NW, NCH_E, CH_E)
    w2 = w_pad.reshape(NW, NCH_E, CH_E)
    d3 = d_pad.reshape(NW, NCH_E, CH_E)

    y0, r0 = _mm1(x, W2_0, W_root0, b0.reshape(1, D))
    p0 = _edge_kernel(y0.reshape(N * R, D), g2, w2, d3, zeros)
    y1, r1 = _mm2(r0, p0[0, :N], p0[1, :N], W2_1, W_root1, b1.reshape(1, D))
    p1 = _edge_kernel(y1.reshape(N * R, D), g2, w2, d3, zeros)
    gvec = _fin(r1, p1[0, :N], p1[1, :N])
    return _head(gvec, lin_ctg_W, lin_ctg_b.reshape(1, -1), embed,
                 dan1_W, dan1_b.reshape(1, -1), msg,
                 out_W.reshape(3, 768, 2), out_b.reshape(1, 2))


def kernel(x, edge_index, edge_type, edge_attr, embed, msg,
           W_rel0, W_root0, b0, W_rel1, W_root1, b1,
           lin_ctg_W, lin_ctg_b, dan1_W, dan1_b, out_W, out_b):
    return _run(x, edge_index, edge_type, embed, msg,
                W_rel0, W_root0, b0, W_rel1, W_root1, b1,
                lin_ctg_W, lin_ctg_b, dan1_W, dan1_b, out_W, out_b)
